# Initial kernel scaffold; baseline (speedup 1.0000x reference)
import jax
import jax.numpy as jnp
from jax.experimental import pallas as pl


def kernel(x, W, n):
    raise NotImplementedError("write your pallas kernel here")


# speedup vs baseline: 109.0422x; 109.0422x over previous
import functools

import numpy as np
import jax
import jax.numpy as jnp
from jax import lax
from jax.experimental import pallas as pl
from jax.experimental.pallas import tpu as pltpu
from jax.experimental.pallas import tpu_sc as plsc

NC = 2
NS = 16
NW = NC * NS
L = 16

B, C, S = 16, 128, 4096
NP = 21
ROWS = B * C
RPW = ROWS // NW
NV = S // L
NB = 256
NHV = NB // L

_POS_XOR = np.int32(0x7FFFFFFF)
_BYTE = np.int32(0xFF)


def _desc_key(u):
  negm = lax.shift_right_arithmetic(u, 31)
  return lax.bitwise_xor(u, lax.bitwise_and(lax.bitwise_not(negm), _POS_XOR))


def _body(x_hbm, w_hbm, n_hbm, out_hbm, perm_hbm,
          xbuf, kbuf0, kbuf1, ibuf0, ibuf1, histA, histB,
          wv, nv, ilbuf, frbuf, fcbuf, outv):
  cid = lax.axis_index("c")
  sid = lax.axis_index("s")
  wid = sid * NC + cid
  row0 = wid * RPW
  b = row0 // C
  c0 = row0 % C

  pltpu.sync_copy(n_hbm, nv)
  pltpu.sync_copy(w_hbm.at[pl.ds(c0 * NP, RPW * NP)], wv)

  lane = lax.iota(jnp.int32, L)
  n_b = jnp.sum(jnp.where(lane == b, nv[...], 0))
  total = jnp.maximum(n_b.astype(jnp.float32) - 1.0, 1.0)

  def prec(j, _):
    o = pl.multiple_of(j * L, L)
    sidx = (lax.iota(jnp.int32, L) + o).astype(jnp.float32)
    t = jnp.minimum(sidx / total, 1.0) * 20.0
    il = t.astype(jnp.int32)
    ilbuf[pl.ds(o, L)] = il
    fr = t - il.astype(jnp.float32)
    frbuf[pl.ds(o, L)] = fr
    fcbuf[pl.ds(o, L)] = 1.0 - fr
    return 0
  lax.fori_loop(0, NV, prec, 0)

  zeros16 = jnp.zeros((L,), jnp.int32)

  def hist_zero(h):
    def z(j, _):
      h[pl.ds(pl.multiple_of(j * L, L), L)] = zeros16
      return 0
    lax.fori_loop(0, NHV, z, 0)

  def hist_scan(h):
    def sc(j, run):
      o = pl.multiple_of(j * L, L)
      hv = h[pl.ds(o, L)]
      cum = plsc.cumsum(hv)
      h[pl.ds(o, L)] = cum - hv + run
      return run + jnp.sum(hv)
    lax.fori_loop(0, NHV, sc, jnp.int32(0))

  def do_row(r, _):
    row = row0 + r
    pltpu.sync_copy(x_hbm.at[row], xbuf)

    hist_zero(histA)

    def tf(j, _):
      o = pl.multiple_of(j * L, L)
      u = plsc.bitcast(xbuf[pl.ds(o, L)], jnp.int32)
      k = _desc_key(u)
      kbuf0[pl.ds(o, L)] = k
      ibuf0[pl.ds(o, L)] = lax.iota(jnp.int32, L) + o
      d = lax.bitwise_and(k, _BYTE)
      cnt, lm = plsc.scan_count(d)
      g = plsc.load_gather(histA, [d])
      plsc.store_scatter(histA, [d], g + cnt, mask=lm)
      return 0
    lax.fori_loop(0, NV, tf, 0)
    hist_scan(histA)

    def permute(ksrc, isrc, kdst, idst, shift, hsrc, next_shift, hdst):
      if hdst is not None:
        hist_zero(hdst)

      def body(j, _):
        o = pl.multiple_of(j * L, L)
        k = ksrc[pl.ds(o, L)]
        iv = isrc[pl.ds(o, L)]
        d = lax.bitwise_and(lax.shift_right_logical(k, shift), _BYTE)
        off = plsc.load_gather(hsrc, [d])
        cnt, lm = plsc.scan_count(d)
        pos = off + cnt - 1
        plsc.store_scatter(kdst, [pos], k)
        plsc.store_scatter(idst, [pos], iv)
        plsc.store_scatter(hsrc, [d], pos + 1, mask=lm)
        if next_shift is not None:
          d2 = lax.bitwise_and(lax.shift_right_logical(k, next_shift), _BYTE)
          cnt2, lm2 = plsc.scan_count(d2)
          g2 = plsc.load_gather(hdst, [d2])
          plsc.store_scatter(hdst, [d2], g2 + cnt2, mask=lm2)
        return 0
      lax.fori_loop(0, NV, body, 0, unroll=2)
      if hdst is not None:
        hist_scan(hdst)

    permute(kbuf0, ibuf0, kbuf1, ibuf1, 0, histA, 8, histB)
    permute(kbuf1, ibuf1, kbuf0, ibuf0, 8, histB, 16, histA)
    permute(kbuf0, ibuf0, kbuf1, ibuf1, 16, histA, 24, histB)
    permute(kbuf1, ibuf1, kbuf0, ibuf0, 24, histB, None, None)

    wbase = r * NP

    def fs(j, acc):
      o = pl.multiple_of(j * L, L)
      k = kbuf0[pl.ds(o, L)]
      v = plsc.bitcast(_desc_key(k), jnp.float32)
      il = ilbuf[pl.ds(o, L)]
      fr = frbuf[pl.ds(o, L)]
      fc = fcbuf[pl.ds(o, L)]
      wl = plsc.load_gather(wv, [il + wbase])
      wr = plsc.load_gather(wv, [jnp.minimum(il + 1, 20) + wbase])
      return acc + v * (fc * wl + fr * wr)
    acc = lax.fori_loop(0, NV, fs, jnp.zeros((L,), jnp.float32), unroll=2)
    val = jnp.sum(acc)
    plsc.store_scatter(outv, [jnp.full((L,), 0, jnp.int32) + r],
                       jnp.full((L,), 0.0, jnp.float32) + val,
                       mask=lane == 0)

    pltpu.sync_copy(ibuf0, perm_hbm.at[row])
    return 0

  lax.fori_loop(0, RPW, do_row, 0)
  pltpu.sync_copy(outv, out_hbm.at[pl.ds(row0, RPW)])


@jax.jit
def _fspool_sc(x2, wflat, n):
  run = pl.kernel(
      _body,
      out_type=(
          jax.ShapeDtypeStruct((ROWS,), jnp.float32),
          jax.ShapeDtypeStruct((ROWS, S), jnp.int32),
      ),
      mesh=plsc.VectorSubcoreMesh(core_axis_name="c", subcore_axis_name="s"),
      scratch_types=[
          pltpu.VMEM((S,), jnp.float32),
          pltpu.VMEM((S,), jnp.int32),
          pltpu.VMEM((S,), jnp.int32),
          pltpu.VMEM((S,), jnp.int32),
          pltpu.VMEM((S,), jnp.int32),
          pltpu.VMEM((NB,), jnp.int32),
          pltpu.VMEM((NB,), jnp.int32),
          pltpu.VMEM((RPW * NP,), jnp.float32),
          pltpu.VMEM((B,), jnp.int32),
          pltpu.VMEM((S,), jnp.int32),
          pltpu.VMEM((S,), jnp.float32),
          pltpu.VMEM((S,), jnp.float32),
          pltpu.VMEM((RPW,), jnp.float32),
      ],
      compiler_params=pltpu.CompilerParams(needs_layout_passes=False),
  )
  return run(x2, wflat, n)


def kernel(x, W, n):
  x2 = x.reshape(ROWS, S)
  out_flat, perm2 = _fspool_sc(x2, W.reshape(-1), n.astype(jnp.int32))
  return out_flat.reshape(B, C), perm2.reshape(B, C, S)


# speedup vs baseline: 122.1043x; 1.1198x over previous
import functools

import numpy as np
import jax
import jax.numpy as jnp
from jax import lax
from jax.experimental import pallas as pl
from jax.experimental.pallas import tpu as pltpu
from jax.experimental.pallas import tpu_sc as plsc

NC = 2
NS = 16
NW = NC * NS
L = 16

B, C, S = 16, 128, 4096
NP = 21
ROWS = B * C
RPW = ROWS // NW
NV = S // L
NB = 2048

_POS_XOR = np.int32(0x7FFFFFFF)
_M11 = np.int32(0x7FF)
_M10 = np.int32(0x3FF)


def _desc_key(u):
  negm = lax.shift_right_arithmetic(u, 31)
  return lax.bitwise_xor(u, lax.bitwise_and(lax.bitwise_not(negm), _POS_XOR))


def _body(x_hbm, w_hbm, n_hbm, out_hbm, perm_hbm,
          xbuf, kbuf0, kbuf1, ibuf0, ibuf1, histA, histB,
          wv, nv, ilbuf, frbuf, fcbuf, outv):
  cid = lax.axis_index("c")
  sid = lax.axis_index("s")
  wid = sid * NC + cid
  row0 = wid * RPW
  b = row0 // C
  c0 = row0 % C

  pltpu.sync_copy(n_hbm, nv)
  pltpu.sync_copy(w_hbm.at[pl.ds(c0 * NP, RPW * NP)], wv)

  lane = lax.iota(jnp.int32, L)
  n_b = jnp.sum(jnp.where(lane == b, nv[...], 0))
  total = jnp.maximum(n_b.astype(jnp.float32) - 1.0, 1.0)

  def prec(j, _):
    o = pl.multiple_of(j * L, L)
    sidx = (lax.iota(jnp.int32, L) + o).astype(jnp.float32)
    t = jnp.minimum(sidx / total, 1.0) * 20.0
    il = t.astype(jnp.int32)
    ilbuf[pl.ds(o, L)] = il
    fr = t - il.astype(jnp.float32)
    frbuf[pl.ds(o, L)] = fr
    fcbuf[pl.ds(o, L)] = 1.0 - fr
    return 0
  lax.fori_loop(0, NV, prec, 0)

  zeros16 = jnp.zeros((L,), jnp.int32)

  def hist_zero(h, nbins):
    def z(j, _):
      h[pl.ds(pl.multiple_of(j * L, L), L)] = zeros16
      return 0
    lax.fori_loop(0, nbins // L, z, 0)

  def hist_scan(h, nbins):
    def sc(j, run):
      o = pl.multiple_of(j * L, L)
      hv = h[pl.ds(o, L)]
      cum = plsc.cumsum(hv)
      h[pl.ds(o, L)] = cum - hv + run
      return run + jnp.sum(hv)
    lax.fori_loop(0, nbins // L, sc, jnp.int32(-1))

  def do_row(r, _):
    row = row0 + r
    pltpu.sync_copy(x_hbm.at[row], xbuf)

    hist_zero(histA, NB)

    def tf(j, _):
      o = pl.multiple_of(j * L, L)
      u = plsc.bitcast(xbuf[pl.ds(o, L)], jnp.int32)
      k = _desc_key(u)
      kbuf0[pl.ds(o, L)] = k
      ibuf0[pl.ds(o, L)] = lax.iota(jnp.int32, L) + o
      d = lax.bitwise_and(k, _M11)
      cnt, lm = plsc.scan_count(d)
      g = plsc.load_gather(histA, [d])
      plsc.store_scatter(histA, [d], g + cnt, mask=lm)
      return 0
    lax.fori_loop(0, NV, tf, 0, unroll=2)
    hist_scan(histA, NB)

    def permute(ksrc, isrc, kdst, idst, shift, mask, hsrc,
                next_shift, next_mask, hdst, next_bins):
      if hdst is not None:
        hist_zero(hdst, next_bins)

      def body(j, _):
        o = pl.multiple_of(j * L, L)
        k = ksrc[pl.ds(o, L)]
        iv = isrc[pl.ds(o, L)]
        d = lax.bitwise_and(lax.shift_right_logical(k, shift), mask)
        off = plsc.load_gather(hsrc, [d])
        cnt, lm = plsc.scan_count(d)
        pos = off + cnt
        plsc.store_scatter(kdst, [pos], k)
        plsc.store_scatter(idst, [pos], iv)
        plsc.store_scatter(hsrc, [d], pos, mask=lm)
        if next_shift is not None:
          d2 = lax.bitwise_and(lax.shift_right_logical(k, next_shift),
                               next_mask)
          cnt2, lm2 = plsc.scan_count(d2)
          g2 = plsc.load_gather(hdst, [d2])
          plsc.store_scatter(hdst, [d2], g2 + cnt2, mask=lm2)
        return 0
      lax.fori_loop(0, NV, body, 0, unroll=2)
      if hdst is not None:
        hist_scan(hdst, next_bins)

    permute(kbuf0, ibuf0, kbuf1, ibuf1, 0, _M11, histA, 11, _M11, histB, NB)
    permute(kbuf1, ibuf1, kbuf0, ibuf0, 11, _M11, histB, 22, _M10, histA, 1024)
    permute(kbuf0, ibuf0, kbuf1, ibuf1, 22, _M10, histA, None, None, None, 0)

    wbase = r * NP

    def fs(j, acc):
      o = pl.multiple_of(j * L, L)
      k = kbuf1[pl.ds(o, L)]
      v = plsc.bitcast(_desc_key(k), jnp.float32)
      il = ilbuf[pl.ds(o, L)]
      fr = frbuf[pl.ds(o, L)]
      fc = fcbuf[pl.ds(o, L)]
      wl = plsc.load_gather(wv, [il + wbase])
      wr = plsc.load_gather(wv, [jnp.minimum(il + 1, 20) + wbase])
      return acc + v * (fc * wl + fr * wr)
    acc = lax.fori_loop(0, NV, fs, jnp.zeros((L,), jnp.float32), unroll=2)
    val = jnp.sum(acc)
    plsc.store_scatter(outv, [jnp.full((L,), 0, jnp.int32) + r],
                       jnp.full((L,), 0.0, jnp.float32) + val,
                       mask=lane == 0)

    pltpu.sync_copy(ibuf1, perm_hbm.at[row])
    return 0

  lax.fori_loop(0, RPW, do_row, 0)
  pltpu.sync_copy(outv, out_hbm.at[pl.ds(row0, RPW)])


@jax.jit
def _fspool_sc(x2, wflat, n):
  run = pl.kernel(
      _body,
      out_type=(
          jax.ShapeDtypeStruct((ROWS,), jnp.float32),
          jax.ShapeDtypeStruct((ROWS, S), jnp.int32),
      ),
      mesh=plsc.VectorSubcoreMesh(core_axis_name="c", subcore_axis_name="s"),
      scratch_types=[
          pltpu.VMEM((S,), jnp.float32),
          pltpu.VMEM((S,), jnp.int32),
          pltpu.VMEM((S,), jnp.int32),
          pltpu.VMEM((S,), jnp.int32),
          pltpu.VMEM((S,), jnp.int32),
          pltpu.VMEM((NB,), jnp.int32),
          pltpu.VMEM((NB,), jnp.int32),
          pltpu.VMEM((RPW * NP,), jnp.float32),
          pltpu.VMEM((B,), jnp.int32),
          pltpu.VMEM((S,), jnp.int32),
          pltpu.VMEM((S,), jnp.float32),
          pltpu.VMEM((S,), jnp.float32),
          pltpu.VMEM((RPW,), jnp.float32),
      ],
      compiler_params=pltpu.CompilerParams(needs_layout_passes=False),
  )
  return run(x2, wflat, n)


def kernel(x, W, n):
  x2 = x.reshape(ROWS, S)
  out_flat, perm2 = _fspool_sc(x2, W.reshape(-1), n.astype(jnp.int32))
  return out_flat.reshape(B, C), perm2.reshape(B, C, S)


# speedup vs baseline: 127.7122x; 1.0459x over previous
import functools

import numpy as np
import jax
import jax.numpy as jnp
from jax import lax
from jax.experimental import pallas as pl
from jax.experimental.pallas import tpu as pltpu
from jax.experimental.pallas import tpu_sc as plsc

NC = 2
NS = 16
NW = NC * NS
L = 16

B, C, S = 16, 128, 4096
NP = 21
ROWS = B * C
RPW = ROWS // NW
PAIRS = RPW // 2
NV = S // L
NB = 2048
S2 = 2 * S

_POS_XOR = np.int32(0x7FFFFFFF)
_M11 = np.int32(0x7FF)
_M10 = np.int32(0x3FF)


def _desc_key(u):
  negm = lax.shift_right_arithmetic(u, 31)
  return lax.bitwise_xor(u, lax.bitwise_and(lax.bitwise_not(negm), _POS_XOR))


def _body(x_hbm, w_hbm, n_hbm, out_hbm, perm_hbm,
          xpair, kbuf0, kbuf1, ibuf0, ibuf1,
          hA0, hA1, hB0, hB1, ibase,
          wv, nv, ilbuf, frbuf, fcbuf, outv):
  cid = lax.axis_index("c")
  sid = lax.axis_index("s")
  wid = sid * NC + cid
  row0 = wid * RPW
  prow0 = wid * PAIRS
  b = row0 // C
  c0 = row0 % C

  pltpu.sync_copy(n_hbm, nv)
  pltpu.sync_copy(w_hbm.at[pl.ds(c0 * NP, RPW * NP)], wv)

  lane = lax.iota(jnp.int32, L)
  n_b = jnp.sum(jnp.where(lane == b, nv[...], 0))
  total = jnp.maximum(n_b.astype(jnp.float32) - 1.0, 1.0)

  def prec(j, _):
    o = pl.multiple_of(j * L, L)
    sidx = lane + o
    ibase[pl.ds(o, L)] = sidx
    t = jnp.minimum(sidx.astype(jnp.float32) / total, 1.0) * 20.0
    il = t.astype(jnp.int32)
    ilbuf[pl.ds(o, L)] = il
    fr = t - il.astype(jnp.float32)
    frbuf[pl.ds(o, L)] = fr
    fcbuf[pl.ds(o, L)] = 1.0 - fr
    return 0
  lax.fori_loop(0, NV, prec, 0)

  zeros16 = jnp.zeros((L,), jnp.int32)

  def hists_zero(ha, hb, nbins):
    def z(j, _):
      o = pl.ds(pl.multiple_of(j * L, L), L)
      ha[o] = zeros16
      hb[o] = zeros16
      return 0
    lax.fori_loop(0, nbins // L, z, 0)

  def hists_scan(ha, hb, nbins):
    def sc(j, runs):
      ra, rb = runs
      o = pl.ds(pl.multiple_of(j * L, L), L)
      va = ha[o]
      vb = hb[o]
      ca = plsc.cumsum(va)
      cb = plsc.cumsum(vb)
      ha[o] = ca - va + ra
      hb[o] = cb - vb + rb
      return ra + jnp.sum(va), rb + jnp.sum(vb)
    lax.fori_loop(0, nbins // L, sc, (jnp.int32(-1), jnp.int32(S - 1)))

  def do_pair(rr, _):
    pltpu.sync_copy(x_hbm.at[prow0 + rr], xpair)

    hists_zero(hA0, hA1, NB)

    def tf(j, _):
      o = pl.multiple_of(j * L, L)
      for off, h in ((0, hA0), (S, hA1)):
        u = plsc.bitcast(xpair[pl.ds(o + off, L)], jnp.int32)
        k = _desc_key(u)
        kbuf0[pl.ds(o + off, L)] = k
        d = lax.bitwise_and(k, _M11)
        cnt, lm = plsc.scan_count(d)
        g = plsc.load_gather(h, [d])
        plsc.store_scatter(h, [d], g + cnt, mask=lm)
      return 0
    lax.fori_loop(0, NV, tf, 0)
    hists_scan(hA0, hA1, NB)

    def permute(ksrc, isrc, kdst, idst, shift, mask, hsrc2,
                next_shift, next_mask, hdst2, next_bins):
      if hdst2 is not None:
        hists_zero(hdst2[0], hdst2[1], next_bins)

      def body(j, _):
        o = pl.multiple_of(j * L, L)
        for st in (0, 1):
          off = st * S
          k = ksrc[pl.ds(o + off, L)]
          iv = (ibase if isrc is None else isrc)[
              pl.ds(o + (0 if isrc is None else off), L)]
          d = lax.bitwise_and(lax.shift_right_logical(k, shift), mask)
          goff = plsc.load_gather(hsrc2[st], [d])
          cnt, lm = plsc.scan_count(d)
          pos = goff + cnt
          plsc.store_scatter(kdst, [pos], k)
          plsc.store_scatter(idst, [pos], iv)
          plsc.store_scatter(hsrc2[st], [d], pos, mask=lm)
          if next_shift is not None:
            d2 = lax.bitwise_and(lax.shift_right_logical(k, next_shift),
                                 next_mask)
            cnt2, lm2 = plsc.scan_count(d2)
            g2 = plsc.load_gather(hdst2[st], [d2])
            plsc.store_scatter(hdst2[st], [d2], g2 + cnt2, mask=lm2)
        return 0
      lax.fori_loop(0, NV, body, 0)
      if hdst2 is not None:
        hists_scan(hdst2[0], hdst2[1], next_bins)

    permute(kbuf0, None, kbuf1, ibuf1, 0, _M11, (hA0, hA1),
            11, _M11, (hB0, hB1), NB)
    permute(kbuf1, ibuf1, kbuf0, ibuf0, 11, _M11, (hB0, hB1),
            22, _M10, (hA0, hA1), 1024)
    permute(kbuf0, ibuf0, kbuf1, ibuf1, 22, _M10, (hA0, hA1),
            None, None, None, 0)

    wbA = (2 * rr) * NP
    wbB = (2 * rr + 1) * NP

    def fs(j, accs):
      accA, accB = accs
      o = pl.multiple_of(j * L, L)
      il = ilbuf[pl.ds(o, L)]
      fr = frbuf[pl.ds(o, L)]
      fc = fcbuf[pl.ds(o, L)]
      ilp = jnp.minimum(il + 1, 20)
      out = []
      for off, wb in ((0, wbA), (S, wbB)):
        k = kbuf1[pl.ds(o + off, L)]
        v = plsc.bitcast(_desc_key(k), jnp.float32)
        wl = plsc.load_gather(wv, [il + wb])
        wr = plsc.load_gather(wv, [ilp + wb])
        out.append(v * (fc * wl + fr * wr))
      return accA + out[0], accB + out[1]
    accA, accB = lax.fori_loop(
        0, NV, fs, (jnp.zeros((L,), jnp.float32), jnp.zeros((L,), jnp.float32)))
    valA = jnp.sum(accA)
    valB = jnp.sum(accB)
    vals = jnp.where(lane == 0, valA, valB)
    plsc.store_scatter(outv, [jnp.minimum(lane, 1) + 2 * rr], vals,
                       mask=lane < 2)

    pltpu.sync_copy(ibuf1, perm_hbm.at[prow0 + rr])
    return 0

  lax.fori_loop(0, PAIRS, do_pair, 0)
  pltpu.sync_copy(outv, out_hbm.at[pl.ds(row0, RPW)])


@jax.jit
def _fspool_sc(x2, wflat, n):
  run = pl.kernel(
      _body,
      out_type=(
          jax.ShapeDtypeStruct((ROWS,), jnp.float32),
          jax.ShapeDtypeStruct((ROWS // 2, S2), jnp.int32),
      ),
      mesh=plsc.VectorSubcoreMesh(core_axis_name="c", subcore_axis_name="s"),
      scratch_types=[
          pltpu.VMEM((S2,), jnp.float32),
          pltpu.VMEM((S2,), jnp.int32),
          pltpu.VMEM((S2,), jnp.int32),
          pltpu.VMEM((S2,), jnp.int32),
          pltpu.VMEM((S2,), jnp.int32),
          pltpu.VMEM((NB,), jnp.int32),
          pltpu.VMEM((NB,), jnp.int32),
          pltpu.VMEM((NB,), jnp.int32),
          pltpu.VMEM((NB,), jnp.int32),
          pltpu.VMEM((S,), jnp.int32),
          pltpu.VMEM((RPW * NP,), jnp.float32),
          pltpu.VMEM((B,), jnp.int32),
          pltpu.VMEM((S,), jnp.int32),
          pltpu.VMEM((S,), jnp.float32),
          pltpu.VMEM((S,), jnp.float32),
          pltpu.VMEM((RPW,), jnp.float32),
      ],
      compiler_params=pltpu.CompilerParams(needs_layout_passes=False),
  )
  return run(x2, wflat, n)


def kernel(x, W, n):
  x2 = x.reshape(ROWS // 2, S2)
  out_flat, perm2 = _fspool_sc(x2, W.reshape(-1), n.astype(jnp.int32))
  return out_flat.reshape(B, C), perm2.reshape(B, C, S)


# speedup vs baseline: 198.2966x; 1.5527x over previous
import functools

import numpy as np
import jax
import jax.numpy as jnp
from jax import lax
from jax.experimental import pallas as pl
from jax.experimental.pallas import tpu as pltpu
from jax.experimental.pallas import tpu_sc as plsc

NC = 2
NS = 16
NW = NC * NS
L = 16

B, C, S = 16, 128, 4096
NP = 21
ROWS = B * C
RPW = ROWS // NW
PAIRS = RPW // 2
NV = S // L
NB = 2048
S2 = 2 * S

_POS_XOR = np.int32(0x7FFFFFFF)
_M11 = np.int32(0x7FF)
_M10 = np.int32(0x3FF)


def _desc_key(u):
  negm = lax.shift_right_arithmetic(u, 31)
  return lax.bitwise_xor(u, lax.bitwise_and(lax.bitwise_not(negm), _POS_XOR))


def _body(x_hbm, w_hbm, n_hbm, out_hbm, perm_hbm,
          xpair, kbuf0, kbuf1, ibuf0, ibuf1,
          hA0, hA1, hB0, hB1, ibase,
          wv, nv, ilbuf, frbuf, fcbuf, outv):
  cid = lax.axis_index("c")
  sid = lax.axis_index("s")
  wid = sid * NC + cid
  row0 = wid * RPW
  prow0 = wid * PAIRS
  b = row0 // C
  c0 = row0 % C

  pltpu.sync_copy(n_hbm, nv)
  pltpu.sync_copy(w_hbm.at[pl.ds(c0 * NP, RPW * NP)], wv)

  lane = lax.iota(jnp.int32, L)
  n_b = jnp.sum(jnp.where(lane == b, nv[...], 0))
  total = jnp.maximum(n_b.astype(jnp.float32) - 1.0, 1.0)

  def prec(j, _):
    o = pl.multiple_of(j * L, L)
    sidx = lane + o
    ibase[pl.ds(o, L)] = sidx
    t = jnp.minimum(sidx.astype(jnp.float32) / total, 1.0) * 20.0
    il = t.astype(jnp.int32)
    ilbuf[pl.ds(o, L)] = il
    fr = t - il.astype(jnp.float32)
    frbuf[pl.ds(o, L)] = fr
    fcbuf[pl.ds(o, L)] = 1.0 - fr
    return 0
  lax.fori_loop(0, NV, prec, 0)

  zeros16 = jnp.zeros((L,), jnp.int32)

  def hists_zero(ha, hb, nbins):
    def z(j, _):
      o = pl.ds(pl.multiple_of(j * L, L), L)
      ha[o] = zeros16
      hb[o] = zeros16
      return 0
    lax.fori_loop(0, nbins // L, z, 0)

  def hists_scan(ha, hb, nbins):
    def sc(j, runs):
      ra, rb = runs
      o = pl.ds(pl.multiple_of(j * L, L), L)
      va = ha[o]
      vb = hb[o]
      ca = plsc.cumsum(va)
      cb = plsc.cumsum(vb)
      ha[o] = ca - va + ra
      hb[o] = cb - vb + rb
      return ra + jnp.sum(va), rb + jnp.sum(vb)
    lax.fori_loop(0, nbins // L, sc, (jnp.int32(-1), jnp.int32(S - 1)))

  def do_pair(rr, _):
    pltpu.sync_copy(x_hbm.at[prow0 + rr], xpair)

    hists_zero(hA0, hA1, NB)

    def tf(j, _):
      o = pl.multiple_of(j * L, L)
      uA = plsc.bitcast(xpair[pl.ds(o, L)], jnp.int32)
      uB = plsc.bitcast(xpair[pl.ds(o + S, L)], jnp.int32)
      kA = _desc_key(uA)
      kB = _desc_key(uB)
      kbuf0[pl.ds(o, L)] = kA
      kbuf0[pl.ds(o + S, L)] = kB
      dA = lax.bitwise_and(kA, _M11)
      dB = lax.bitwise_and(kB, _M11)
      cntA, lmA = plsc.scan_count(dA)
      cntB, lmB = plsc.scan_count(dB)
      gA = plsc.load_gather(hA0, [dA])
      gB = plsc.load_gather(hA1, [dB])
      plsc.store_scatter(hA0, [dA], gA + cntA, mask=lmA)
      plsc.store_scatter(hA1, [dB], gB + cntB, mask=lmB)
      return 0
    lax.fori_loop(0, NV, tf, 0)
    hists_scan(hA0, hA1, NB)

    def permute(ksrc, isrc, kdst, idst, shift, mask, hsrc2,
                next_shift, next_mask, hdst2, next_bins):
      if hdst2 is not None:
        hists_zero(hdst2[0], hdst2[1], next_bins)

      def body(j, _):
        o = pl.multiple_of(j * L, L)
        kA = ksrc[pl.ds(o, L)]
        kB = ksrc[pl.ds(o + S, L)]
        if isrc is None:
          ivA = ivB = ibase[pl.ds(o, L)]
        else:
          ivA = isrc[pl.ds(o, L)]
          ivB = isrc[pl.ds(o + S, L)]
        dA = lax.bitwise_and(lax.shift_right_logical(kA, shift), mask)
        dB = lax.bitwise_and(lax.shift_right_logical(kB, shift), mask)
        cntA, lmA = plsc.scan_count(dA)
        cntB, lmB = plsc.scan_count(dB)
        offA = plsc.load_gather(hsrc2[0], [dA])
        offB = plsc.load_gather(hsrc2[1], [dB])
        posA = offA + cntA
        posB = offB + cntB
        plsc.store_scatter(kdst, [posA], kA)
        plsc.store_scatter(kdst, [posB], kB)
        plsc.store_scatter(idst, [posA], ivA)
        plsc.store_scatter(idst, [posB], ivB)
        plsc.store_scatter(hsrc2[0], [dA], posA, mask=lmA)
        plsc.store_scatter(hsrc2[1], [dB], posB, mask=lmB)
        if next_shift is not None:
          d2A = lax.bitwise_and(lax.shift_right_logical(kA, next_shift),
                                next_mask)
          d2B = lax.bitwise_and(lax.shift_right_logical(kB, next_shift),
                                next_mask)
          cnt2A, lm2A = plsc.scan_count(d2A)
          cnt2B, lm2B = plsc.scan_count(d2B)
          g2A = plsc.load_gather(hdst2[0], [d2A])
          g2B = plsc.load_gather(hdst2[1], [d2B])
          plsc.store_scatter(hdst2[0], [d2A], g2A + cnt2A, mask=lm2A)
          plsc.store_scatter(hdst2[1], [d2B], g2B + cnt2B, mask=lm2B)
        return 0
      lax.fori_loop(0, NV, body, 0)
      if hdst2 is not None:
        hists_scan(hdst2[0], hdst2[1], next_bins)

    permute(kbuf0, None, kbuf1, ibuf1, 0, _M11, (hA0, hA1),
            11, _M11, (hB0, hB1), NB)
    permute(kbuf1, ibuf1, kbuf0, ibuf0, 11, _M11, (hB0, hB1),
            22, _M10, (hA0, hA1), 1024)
    permute(kbuf0, ibuf0, kbuf1, ibuf1, 22, _M10, (hA0, hA1),
            None, None, None, 0)

    wbA = (2 * rr) * NP
    wbB = (2 * rr + 1) * NP

    def fs(j, accs):
      accA, accB = accs
      o = pl.multiple_of(j * L, L)
      kA = kbuf1[pl.ds(o, L)]
      kB = kbuf1[pl.ds(o + S, L)]
      il = ilbuf[pl.ds(o, L)]
      fr = frbuf[pl.ds(o, L)]
      fc = fcbuf[pl.ds(o, L)]
      ilp = jnp.minimum(il + 1, 20)
      vA = plsc.bitcast(_desc_key(kA), jnp.float32)
      vB = plsc.bitcast(_desc_key(kB), jnp.float32)
      wlA = plsc.load_gather(wv, [il + wbA])
      wlB = plsc.load_gather(wv, [il + wbB])
      wrA = plsc.load_gather(wv, [ilp + wbA])
      wrB = plsc.load_gather(wv, [ilp + wbB])
      return (accA + vA * (fc * wlA + fr * wrA),
              accB + vB * (fc * wlB + fr * wrB))
    accA, accB = lax.fori_loop(
        0, NV, fs, (jnp.zeros((L,), jnp.float32), jnp.zeros((L,), jnp.float32)))
    valA = jnp.sum(accA)
    valB = jnp.sum(accB)
    vals = jnp.where(lane == 0, valA, valB)
    plsc.store_scatter(outv, [jnp.minimum(lane, 1) + 2 * rr], vals,
                       mask=lane < 2)

    pltpu.sync_copy(ibuf1, perm_hbm.at[prow0 + rr])
    return 0

  lax.fori_loop(0, PAIRS, do_pair, 0)
  pltpu.sync_copy(outv, out_hbm.at[pl.ds(row0, RPW)])


@jax.jit
def _fspool_sc(x2, wflat, n):
  run = pl.kernel(
      _body,
      out_type=(
          jax.ShapeDtypeStruct((ROWS,), jnp.float32),
          jax.ShapeDtypeStruct((ROWS // 2, S2), jnp.int32),
      ),
      mesh=plsc.VectorSubcoreMesh(core_axis_name="c", subcore_axis_name="s"),
      scratch_types=[
          pltpu.VMEM((S2,), jnp.float32),
          pltpu.VMEM((S2,), jnp.int32),
          pltpu.VMEM((S2,), jnp.int32),
          pltpu.VMEM((S2,), jnp.int32),
          pltpu.VMEM((S2,), jnp.int32),
          pltpu.VMEM((NB,), jnp.int32),
          pltpu.VMEM((NB,), jnp.int32),
          pltpu.VMEM((NB,), jnp.int32),
          pltpu.VMEM((NB,), jnp.int32),
          pltpu.VMEM((S,), jnp.int32),
          pltpu.VMEM((RPW * NP,), jnp.float32),
          pltpu.VMEM((B,), jnp.int32),
          pltpu.VMEM((S,), jnp.int32),
          pltpu.VMEM((S,), jnp.float32),
          pltpu.VMEM((S,), jnp.float32),
          pltpu.VMEM((RPW,), jnp.float32),
      ],
      compiler_params=pltpu.CompilerParams(needs_layout_passes=False),
  )
  return run(x2, wflat, n)


def kernel(x, W, n):
  x2 = x.reshape(ROWS // 2, S2)
  out_flat, perm2 = _fspool_sc(x2, W.reshape(-1), n.astype(jnp.int32))
  return out_flat.reshape(B, C), perm2.reshape(B, C, S)


# speedup vs baseline: 311.9889x; 1.5733x over previous
import functools

import numpy as np
import jax
import jax.numpy as jnp
from jax import lax
from jax.experimental import pallas as pl
from jax.experimental.pallas import tpu as pltpu
from jax.experimental.pallas import tpu_sc as plsc

NC = 2
NS = 16
NW = NC * NS
L = 16

B, C, S = 16, 128, 4096
NP = 21
ROWS = B * C
RPW = ROWS // NW
T = 4
GROUPS = RPW // T
NV = S // L
NB = 2048
SG = T * S

_POS_XOR = np.int32(0x7FFFFFFF)
_M11 = np.int32(0x7FF)
_M10 = np.int32(0x3FF)


def _desc_key(u):
  negm = lax.shift_right_arithmetic(u, 31)
  return lax.bitwise_xor(u, lax.bitwise_and(lax.bitwise_not(negm), _POS_XOR))


def _body(x_hbm, w_hbm, n_hbm, out_hbm, perm_hbm, *scr):
  xg, kbuf0, kbuf1, ibuf0, ibuf1 = scr[:5]
  hA = scr[5:5 + T]
  hB = scr[5 + T:5 + 2 * T]
  ibase, wv, nv, ilbuf, frbuf, fcbuf, outv = scr[5 + 2 * T:]

  cid = lax.axis_index("c")
  sid = lax.axis_index("s")
  wid = sid * NC + cid
  row0 = wid * RPW
  grp0 = wid * GROUPS
  b = row0 // C
  c0 = row0 % C

  pltpu.sync_copy(n_hbm, nv)
  pltpu.sync_copy(w_hbm.at[pl.ds(c0 * NP, RPW * NP)], wv)

  lane = lax.iota(jnp.int32, L)
  n_b = jnp.sum(jnp.where(lane == b, nv[...], 0))
  total = jnp.maximum(n_b.astype(jnp.float32) - 1.0, 1.0)

  def prec(j, _):
    o = pl.multiple_of(j * L, L)
    sidx = lane + o
    ibase[pl.ds(o, L)] = sidx
    t = jnp.minimum(sidx.astype(jnp.float32) / total, 1.0) * 20.0
    il = t.astype(jnp.int32)
    ilbuf[pl.ds(o, L)] = il
    fr = t - il.astype(jnp.float32)
    frbuf[pl.ds(o, L)] = fr
    fcbuf[pl.ds(o, L)] = 1.0 - fr
    return 0
  lax.fori_loop(0, NV, prec, 0)

  zeros16 = jnp.zeros((L,), jnp.int32)

  def hists_zero(hs, nbins):
    def z(j, _):
      o = pl.ds(pl.multiple_of(j * L, L), L)
      for h in hs:
        h[o] = zeros16
      return 0
    lax.fori_loop(0, nbins // L, z, 0)

  def hists_scan(hs, nbins):
    def sc(j, runs):
      o = pl.ds(pl.multiple_of(j * L, L), L)
      vs = [h[o] for h in hs]
      cs = [plsc.cumsum(v) for v in vs]
      for h, v, cum, run in zip(hs, vs, cs, runs):
        h[o] = cum - v + run
      return tuple(run + jnp.sum(v) for run, v in zip(runs, vs))
    lax.fori_loop(0, nbins // L, sc,
                  tuple(jnp.int32(t * S - 1) for t in range(T)))

  def do_group(rr, _):
    pltpu.sync_copy(x_hbm.at[grp0 + rr], xg)

    hists_zero(hA, NB)

    def tf(j, _):
      o = pl.multiple_of(j * L, L)
      us = [plsc.bitcast(xg[pl.ds(o + t * S, L)], jnp.int32)
            for t in range(T)]
      ks = [_desc_key(u) for u in us]
      for t in range(T):
        kbuf0[pl.ds(o + t * S, L)] = ks[t]
      ds = [lax.bitwise_and(k, _M11) for k in ks]
      sc = [plsc.scan_count(d) for d in ds]
      gs = [plsc.load_gather(h, [d]) for h, d in zip(hA, ds)]
      for t in range(T):
        plsc.store_scatter(hA[t], [ds[t]], gs[t] + sc[t][0], mask=sc[t][1])
      return 0
    lax.fori_loop(0, NV, tf, 0)
    hists_scan(hA, NB)

    def permute(ksrc, isrc, kdst, idst, shift, mask, hsrc,
                next_shift, next_mask, hdst, next_bins):
      if hdst is not None:
        hists_zero(hdst, next_bins)

      def body(j, _):
        o = pl.multiple_of(j * L, L)
        ks = [ksrc[pl.ds(o + t * S, L)] for t in range(T)]
        if isrc is None:
          iv0 = ibase[pl.ds(o, L)]
          ivs = [iv0] * T
        else:
          ivs = [isrc[pl.ds(o + t * S, L)] for t in range(T)]
        ds = [lax.bitwise_and(lax.shift_right_logical(k, shift), mask)
              for k in ks]
        sc = [plsc.scan_count(d) for d in ds]
        offs = [plsc.load_gather(h, [d]) for h, d in zip(hsrc, ds)]
        poss = [off + cnt for off, (cnt, _) in zip(offs, sc)]
        for t in range(T):
          plsc.store_scatter(kdst, [poss[t]], ks[t])
        for t in range(T):
          plsc.store_scatter(idst, [poss[t]], ivs[t])
        for t in range(T):
          plsc.store_scatter(hsrc[t], [ds[t]], poss[t], mask=sc[t][1])
        if next_shift is not None:
          d2 = [lax.bitwise_and(lax.shift_right_logical(k, next_shift),
                                next_mask) for k in ks]
          sc2 = [plsc.scan_count(d) for d in d2]
          g2 = [plsc.load_gather(h, [d]) for h, d in zip(hdst, d2)]
          for t in range(T):
            plsc.store_scatter(hdst[t], [d2[t]], g2[t] + sc2[t][0],
                               mask=sc2[t][1])
        return 0
      lax.fori_loop(0, NV, body, 0)
      if hdst is not None:
        hists_scan(hdst, next_bins)

    permute(kbuf0, None, kbuf1, ibuf1, 0, _M11, hA, 11, _M11, hB, NB)
    permute(kbuf1, ibuf1, kbuf0, ibuf0, 11, _M11, hB, 22, _M10, hA, 1024)
    permute(kbuf0, ibuf0, kbuf1, ibuf1, 22, _M10, hA, None, None, None, 0)

    wbs = [(T * rr + t) * NP for t in range(T)]

    def fs(j, accs):
      o = pl.multiple_of(j * L, L)
      ks = [kbuf1[pl.ds(o + t * S, L)] for t in range(T)]
      il = ilbuf[pl.ds(o, L)]
      fr = frbuf[pl.ds(o, L)]
      fc = fcbuf[pl.ds(o, L)]
      ilp = jnp.minimum(il + 1, 20)
      vs = [plsc.bitcast(_desc_key(k), jnp.float32) for k in ks]
      wls = [plsc.load_gather(wv, [il + wb]) for wb in wbs]
      wrs = [plsc.load_gather(wv, [ilp + wb]) for wb in wbs]
      return tuple(acc + v * (fc * wl + fr * wr)
                   for acc, v, wl, wr in zip(accs, vs, wls, wrs))
    accs = lax.fori_loop(0, NV, fs,
                         tuple(jnp.zeros((L,), jnp.float32)
                               for _ in range(T)))
    vals = [jnp.sum(a) for a in accs]
    sel = vals[T - 1]
    for t in range(T - 2, -1, -1):
      sel = jnp.where(lane == t, vals[t], sel)
    plsc.store_scatter(outv, [jnp.minimum(lane, T - 1) + T * rr], sel,
                       mask=lane < T)

    pltpu.sync_copy(ibuf1, perm_hbm.at[grp0 + rr])
    return 0

  lax.fori_loop(0, GROUPS, do_group, 0)
  pltpu.sync_copy(outv, out_hbm.at[pl.ds(row0, RPW)])


@jax.jit
def _fspool_sc(x2, wflat, n):
  run = pl.kernel(
      _body,
      out_type=(
          jax.ShapeDtypeStruct((ROWS,), jnp.float32),
          jax.ShapeDtypeStruct((ROWS // T, SG), jnp.int32),
      ),
      mesh=plsc.VectorSubcoreMesh(core_axis_name="c", subcore_axis_name="s"),
      scratch_types=(
          [
              pltpu.VMEM((SG,), jnp.float32),
              pltpu.VMEM((SG,), jnp.int32),
              pltpu.VMEM((SG,), jnp.int32),
              pltpu.VMEM((SG,), jnp.int32),
              pltpu.VMEM((SG,), jnp.int32),
          ]
          + [pltpu.VMEM((NB,), jnp.int32) for _ in range(2 * T)]
          + [
              pltpu.VMEM((S,), jnp.int32),
              pltpu.VMEM((RPW * NP,), jnp.float32),
              pltpu.VMEM((B,), jnp.int32),
              pltpu.VMEM((S,), jnp.int32),
              pltpu.VMEM((S,), jnp.float32),
              pltpu.VMEM((S,), jnp.float32),
              pltpu.VMEM((RPW,), jnp.float32),
          ]
      ),
      compiler_params=pltpu.CompilerParams(needs_layout_passes=False),
  )
  return run(x2, wflat, n)


def kernel(x, W, n):
  x2 = x.reshape(ROWS // T, SG)
  out_flat, perm2 = _fspool_sc(x2, W.reshape(-1), n.astype(jnp.int32))
  return out_flat.reshape(B, C), perm2.reshape(B, C, S)
